# Initial kernel scaffold; baseline (speedup 1.0000x reference)
import jax
import jax.numpy as jnp
from jax.experimental import pallas as pl


def kernel(h, edge_index):
    raise NotImplementedError("write your pallas kernel here")


# speedup vs baseline: 1.2308x; 1.2308x over previous
import functools

import jax
import jax.numpy as jnp
from jax import lax
from jax.experimental import pallas as pl
from jax.experimental.pallas import tpu as pltpu
from jax.experimental.pallas import tpu_sc as plsc

N_NODES = 10000
N_EDGES = 320000
D_FEAT = 128
NUM_CORES = 2
NUM_SUBCORES = 16
NW = NUM_CORES * NUM_SUBCORES
EDGES_PER_W = N_EDGES // NW
CHUNK = 80
NUM_CHUNKS = EDGES_PER_W // CHUNK


def _dot_scores(h, src, dst):
    mesh = plsc.VectorSubcoreMesh(core_axis_name="c", subcore_axis_name="s")

    @functools.partial(
        pl.kernel,
        out_type=jax.ShapeDtypeStruct((N_EDGES,), jnp.float32),
        mesh=mesh,
        compiler_params=pltpu.CompilerParams(needs_layout_passes=False),
        scratch_types=[
            pltpu.VMEM((CHUNK,), jnp.int32),
            pltpu.VMEM((CHUNK,), jnp.int32),
            pltpu.VMEM((CHUNK, D_FEAT), jnp.float32),
            pltpu.VMEM((CHUNK, D_FEAT), jnp.float32),
            pltpu.VMEM((CHUNK,), jnp.float32),
            pltpu.SemaphoreType.DMA,
            pltpu.SemaphoreType.DMA,
        ],
    )
    def scores_kernel(h_hbm, src_hbm, dst_hbm, out_hbm,
                      idx_s, idx_d, rows_a, rows_b, scores, sem_a, sem_b):
        wid = lax.axis_index("s") * NUM_CORES + lax.axis_index("c")
        wbase = wid * EDGES_PER_W

        def chunk_body(i, carry):
            base = pl.multiple_of(wbase + i * CHUNK, 8)
            pltpu.sync_copy(src_hbm.at[pl.ds(base, CHUNK)], idx_s)
            pltpu.sync_copy(dst_hbm.at[pl.ds(base, CHUNK)], idx_d)
            cp_a = pltpu.async_copy(h_hbm.at[idx_s], rows_a, sem_a)
            cp_b = pltpu.async_copy(h_hbm.at[idx_d], rows_b, sem_b)
            cp_a.wait()
            cp_b.wait()

            lane = lax.iota(jnp.int32, 16)

            def group_body(g, c):
                rid = g * 16 + lane
                zero = jnp.zeros((16,), jnp.float32)

                def d_body(j, accs):
                    a0, a1, a2, a3 = accs
                    d = j * 4
                    for u in range(4):
                        col = jnp.full((16,), d + u, jnp.int32)
                        va = plsc.load_gather(rows_a, [rid, col])
                        vb = plsc.load_gather(rows_b, [rid, col])
                        if u == 0:
                            a0 = a0 + va * vb
                        elif u == 1:
                            a1 = a1 + va * vb
                        elif u == 2:
                            a2 = a2 + va * vb
                        else:
                            a3 = a3 + va * vb
                    return (a0, a1, a2, a3)

                a0, a1, a2, a3 = lax.fori_loop(
                    0, D_FEAT // 4, d_body, (zero, zero, zero, zero))
                scores[pl.ds(g * 16, 16)] = (a0 + a1) + (a2 + a3)
                return c

            lax.fori_loop(0, CHUNK // 16, group_body, 0)
            pltpu.sync_copy(scores, out_hbm.at[pl.ds(base, CHUNK)])
            return carry

        lax.fori_loop(0, NUM_CHUNKS, chunk_body, 0)

    return scores_kernel(h, src, dst)


def kernel(h, edge_index):
    src = edge_index[0].astype(jnp.int32)
    dst = edge_index[1].astype(jnp.int32)
    return _dot_scores(h, src, dst)
